# baseline (speedup 1.0000x reference)
import functools

import jax
import jax.numpy as jnp
from jax import lax
from jax.experimental import pallas as pl
from jax.experimental.pallas import tpu as pltpu
from jax.experimental.pallas import tpu_sc as plsc

C = 3
T = 300
S = 32
ROW = 256 * 256
NC, NS_SUB = 2, 16
NW = NC * NS_SUB
FRAMES = C * S
FPW = FRAMES // NW


def _sc_subsample(x_flat):
    mesh = plsc.VectorSubcoreMesh(core_axis_name="c", subcore_axis_name="s")

    @functools.partial(
        pl.kernel,
        mesh=mesh,
        out_type=jax.ShapeDtypeStruct((FRAMES * ROW,), jnp.float32),
        scratch_types=[
            pltpu.SemaphoreType.DMA,
        ],
    )
    def k(x_hbm, out_hbm, sem):
        wid = lax.axis_index("s") * NC + lax.axis_index("c")
        f0 = wid * FPW

        copies = []
        for j in range(FPW):
            f = f0 + j
            c = f // S
            i = f - c * S
            src = (i * (T - 1)) // (S - 1)
            copies.append(pltpu.async_copy(
                x_hbm.at[pl.ds(((c * T + src) * ROW), ROW)],
                out_hbm.at[pl.ds(f * ROW, ROW)],
                sem))
        for cp in copies:
            cp.wait()

    return k(x_flat)


def kernel(x):
    x_flat = x.reshape(C * T * ROW)
    out = _sc_subsample(x_flat)
    return out.reshape(C, S, 256, 256)


# speedup vs baseline: 4.1657x; 4.1657x over previous
import functools

import jax
import jax.numpy as jnp
from jax import lax
from jax.experimental import pallas as pl
from jax.experimental.pallas import tpu as pltpu
from jax.experimental.pallas import tpu_sc as plsc

C = 3
T = 300
S = 32
ROW = 256 * 256
CHUNK = ROW // 2
NC, NS_SUB = 2, 16
NW = NC * NS_SUB
FRAMES = C * S
FPW = FRAMES // NW
CPW = FPW * (ROW // CHUNK)
NBUF = 3
NSUB = 8
SUB = CHUNK // NSUB


def _sc_subsample(x_flat):
    mesh = plsc.VectorSubcoreMesh(core_axis_name="c", subcore_axis_name="s")

    @functools.partial(
        pl.kernel,
        mesh=mesh,
        out_type=jax.ShapeDtypeStruct((FRAMES * ROW,), jnp.float32),
        scratch_types=[
            pltpu.VMEM((CHUNK,), jnp.float32),
            pltpu.VMEM((CHUNK,), jnp.float32),
            pltpu.VMEM((CHUNK,), jnp.float32),
            pltpu.SemaphoreType.DMA((NBUF,)),
            pltpu.SemaphoreType.DMA((NBUF,)),
        ],
    )
    def k(x_hbm, out_hbm, buf0, buf1, buf2, rsem, wsem):
        bufs = (buf0, buf1, buf2)
        wid = lax.axis_index("s") * NC + lax.axis_index("c")
        f0 = wid * FPW

        def src_word(kk):
            f = f0 + (kk // 2)
            c = f // S
            i = f - c * S
            src = (i * (T - 1)) // (S - 1)
            return (c * T + src) * ROW + (kk % 2) * CHUNK

        def dst_word(kk):
            f = f0 + (kk // 2)
            return f * ROW + (kk % 2) * CHUNK

        def read(kk):
            base = src_word(kk)
            return [pltpu.async_copy(
                        x_hbm.at[pl.ds(base + q * SUB, SUB)],
                        bufs[kk % NBUF].at[pl.ds(q * SUB, SUB)],
                        rsem.at[kk % NBUF])
                    for q in range(NSUB)]

        def write(kk):
            base = dst_word(kk)
            return [pltpu.async_copy(
                        bufs[kk % NBUF].at[pl.ds(q * SUB, SUB)],
                        out_hbm.at[pl.ds(base + q * SUB, SUB)],
                        wsem.at[kk % NBUF])
                    for q in range(NSUB)]

        def wait_all(group):
            for cp in group:
                cp.wait()

        reads = [None] * CPW
        writes = [None] * CPW
        for kk in range(min(NBUF, CPW)):
            reads[kk] = read(kk)
        for kk in range(CPW):
            wait_all(reads[kk])
            writes[kk] = write(kk)
            nxt = kk + 2
            if kk >= 1 and nxt < CPW:
                wait_all(writes[kk - 1])
                reads[nxt] = read(nxt)
        for kk in range(max(0, CPW - NBUF), CPW):
            if writes[kk] is not None:
                wait_all(writes[kk])

    return k(x_flat)


def kernel(x):
    x_flat = x.reshape(C * T * ROW)
    out = _sc_subsample(x_flat)
    return out.reshape(C, S, 256, 256)


# speedup vs baseline: 23.9273x; 5.7439x over previous
import functools

import jax
import jax.numpy as jnp
from jax import lax
from jax.experimental import pallas as pl
from jax.experimental.pallas import tpu as pltpu
from jax.experimental.pallas import tpu_sc as plsc

C = 3
T = 300
S = 32
H, W = 256, 256
NC, NS_SUB = 2, 16
NW = NC * NS_SUB
FRAMES = C * S
FPW = FRAMES // NW
HCH = H // 2
CPW = FPW * 2
NBUF = 3
NSUB = 8
RSUB = HCH // NSUB


def _sc_subsample(x):
    mesh = plsc.VectorSubcoreMesh(core_axis_name="c", subcore_axis_name="s")

    @functools.partial(
        pl.kernel,
        mesh=mesh,
        out_type=jax.ShapeDtypeStruct((C, S, H, W), jnp.float32),
        scratch_types=[
            pltpu.VMEM((HCH, W), jnp.float32),
            pltpu.VMEM((HCH, W), jnp.float32),
            pltpu.VMEM((HCH, W), jnp.float32),
            pltpu.SemaphoreType.DMA((NBUF,)),
            pltpu.SemaphoreType.DMA((NBUF,)),
        ],
    )
    def k(x_hbm, out_hbm, buf0, buf1, buf2, rsem, wsem):
        bufs = (buf0, buf1, buf2)
        wid = lax.axis_index("s") * NC + lax.axis_index("c")
        f0 = wid * FPW

        def coords(kk):
            f = f0 + (kk // 2)
            c = f // S
            i = f - c * S
            src = (i * (T - 1)) // (S - 1)
            return c, i, src, (kk % 2) * HCH

        def read(kk):
            c, _, src, r0 = coords(kk)
            return [pltpu.async_copy(
                        x_hbm.at[c, src, pl.ds(r0 + q * RSUB, RSUB), :],
                        bufs[kk % NBUF].at[pl.ds(q * RSUB, RSUB), :],
                        rsem.at[kk % NBUF])
                    for q in range(NSUB)]

        def write(kk):
            c, i, _, r0 = coords(kk)
            return [pltpu.async_copy(
                        bufs[kk % NBUF].at[pl.ds(q * RSUB, RSUB), :],
                        out_hbm.at[c, i, pl.ds(r0 + q * RSUB, RSUB), :],
                        wsem.at[kk % NBUF])
                    for q in range(NSUB)]

        def wait_all(group):
            for cp in group:
                cp.wait()

        reads = [None] * CPW
        writes = [None] * CPW
        for kk in range(min(NBUF, CPW)):
            reads[kk] = read(kk)
        for kk in range(CPW):
            wait_all(reads[kk])
            writes[kk] = write(kk)
            nxt = kk + 2
            if kk >= 1 and nxt < CPW:
                wait_all(writes[kk - 1])
                reads[nxt] = read(nxt)
        for kk in range(max(0, CPW - NBUF), CPW):
            if writes[kk] is not None:
                wait_all(writes[kk])

    return k(x)


def kernel(x):
    return _sc_subsample(x)


# speedup vs baseline: 26.2520x; 1.0972x over previous
import functools

import jax
import jax.numpy as jnp
from jax import lax
from jax.experimental import pallas as pl
from jax.experimental.pallas import tpu as pltpu
from jax.experimental.pallas import tpu_sc as plsc

C = 3
T = 300
S = 32
H, W = 256, 256
NC, NS_SUB = 2, 16
NW = NC * NS_SUB
FRAMES = C * S
FPW = FRAMES // NW
HCH = H // 2
CPW = FPW * 2
NBUF = 3
NSUB = 1
RSUB = HCH // NSUB


def _sc_subsample(x):
    mesh = plsc.VectorSubcoreMesh(core_axis_name="c", subcore_axis_name="s")

    @functools.partial(
        pl.kernel,
        mesh=mesh,
        out_type=jax.ShapeDtypeStruct((C, S, H, W), jnp.float32),
        scratch_types=[
            pltpu.VMEM((HCH, W), jnp.float32),
            pltpu.VMEM((HCH, W), jnp.float32),
            pltpu.VMEM((HCH, W), jnp.float32),
            pltpu.SemaphoreType.DMA((NBUF,)),
            pltpu.SemaphoreType.DMA((NBUF,)),
        ],
    )
    def k(x_hbm, out_hbm, buf0, buf1, buf2, rsem, wsem):
        bufs = (buf0, buf1, buf2)
        wid = lax.axis_index("s") * NC + lax.axis_index("c")
        f0 = wid * FPW

        def coords(kk):
            f = f0 + (kk // 2)
            c = f // S
            i = f - c * S
            src = (i * (T - 1)) // (S - 1)
            return c, i, src, (kk % 2) * HCH

        def read(kk):
            c, _, src, r0 = coords(kk)
            return [pltpu.async_copy(
                        x_hbm.at[c, src, pl.ds(r0 + q * RSUB, RSUB), :],
                        bufs[kk % NBUF].at[pl.ds(q * RSUB, RSUB), :],
                        rsem.at[kk % NBUF])
                    for q in range(NSUB)]

        def write(kk):
            c, i, _, r0 = coords(kk)
            return [pltpu.async_copy(
                        bufs[kk % NBUF].at[pl.ds(q * RSUB, RSUB), :],
                        out_hbm.at[c, i, pl.ds(r0 + q * RSUB, RSUB), :],
                        wsem.at[kk % NBUF])
                    for q in range(NSUB)]

        def wait_all(group):
            for cp in group:
                cp.wait()

        reads = [None] * CPW
        writes = [None] * CPW
        for kk in range(min(NBUF, CPW)):
            reads[kk] = read(kk)
        for kk in range(CPW):
            wait_all(reads[kk])
            writes[kk] = write(kk)
            nxt = kk + 2
            if kk >= 1 and nxt < CPW:
                wait_all(writes[kk - 1])
                reads[nxt] = read(nxt)
        for kk in range(max(0, CPW - NBUF), CPW):
            if writes[kk] is not None:
                wait_all(writes[kk])

    return k(x)


def kernel(x):
    return _sc_subsample(x)
